# baseline (speedup 1.0000x reference)
import functools

import jax
import jax.numpy as jnp
from jax import lax
from jax.experimental import pallas as pl
from jax.experimental.pallas import tpu as pltpu
from jax.experimental.pallas import tpu_sc as plsc

NUM_FIELD = 39
NUM_CONT = 13
NUM_CAT = NUM_FIELD - NUM_CONT
EMB = 16
BATCH = 16384
LANES = 16

NUM_CORES = 2
NUM_SUBCORES = 16
NW = NUM_CORES * NUM_SUBCORES
ROWS_PER_W = BATCH // NW
CHUNK = 128
NCHUNKS = ROWS_PER_W // CHUNK
GROUPS = CHUNK // LANES
OUT_ROWS = NUM_FIELD * EMB

_mesh = plsc.VectorSubcoreMesh(core_axis_name="c", subcore_axis_name="s")


@functools.partial(
    pl.kernel,
    mesh=_mesh,
    compiler_params=pltpu.CompilerParams(
        needs_layout_passes=False, use_tc_tiling_on_sc=True),
    out_type=jax.ShapeDtypeStruct((OUT_ROWS, BATCH), jnp.float32),
    scratch_types=[
        pltpu.VMEM((OUT_ROWS,), jnp.float32),
        pltpu.VMEM((OUT_ROWS * LANES,), jnp.float32),
        pltpu.VMEM((2 * CHUNK * NUM_FIELD,), jnp.float32),
        pltpu.VMEM((OUT_ROWS, CHUNK), jnp.float32),
        pltpu.VMEM((GROUPS * NUM_CAT * LANES,), jnp.int32),
        pltpu.VMEM((GROUPS * NUM_CAT * LANES,), jnp.int32),
        pltpu.SemaphoreType.DMA,
        pltpu.SemaphoreType.DMA,
    ],
)
def _emb_sc(in_hbm, v_hbm, out_hbm, v_v, vs_v, in_v, out_v, as_v, ms_v,
            in_sem, out_sem):
    wid = lax.axis_index("s") * NUM_CORES + lax.axis_index("c")
    iota = lax.iota(jnp.int32, LANES)
    pltpu.sync_copy(v_hbm, v_v)

    def splat_body(j, _):
        for c in range(EMB):
            sp = plsc.load_gather(
                v_v, [jnp.broadcast_to(j * EMB + c, (LANES,)).astype(jnp.int32)])
            plsc.store_scatter(vs_v, [(j * EMB + c) * LANES + iota], sp)
        return 0

    lax.fori_loop(0, NUM_FIELD, splat_body, 0)

    def in_copy(ci):
        base = wid * ROWS_PER_W + ci * CHUNK
        return pltpu.make_async_copy(
            in_hbm.at[pl.ds(base * NUM_FIELD, CHUNK * NUM_FIELD)],
            in_v.at[pl.ds((ci % 2) * (CHUNK * NUM_FIELD), CHUNK * NUM_FIELD)],
            in_sem)

    def out_copy(ci):
        base = wid * ROWS_PER_W + ci * CHUNK
        return pltpu.make_async_copy(
            out_v,
            out_hbm.at[:, pl.ds(base, CHUNK)],
            out_sem)

    def scatter_16cols(x, rowv, cols_c, j):
        for c0 in (0, 8):
            vss = [vs_v[pl.ds((j * EMB + c) * LANES, LANES)]
                   for c in range(c0, c0 + 8)]
            vals = [x * vs for vs in vss]
            for i, c in enumerate(range(c0, c0 + 8)):
                plsc.store_scatter(out_v, [rowv, cols_c[c]], vals[i])

    in_copy(0).start()

    def chunk_body(ci, _):
        @pl.when(ci + 1 < NCHUNKS)
        def _prefetch():
            in_copy(ci + 1).start()

        in_copy(ci).wait()

        @pl.when(ci >= 1)
        def _drain():
            out_copy(ci - 1).wait()

        ibase = (ci % 2) * (CHUNK * NUM_FIELD)

        def group_body(g, _):
            rows = g * LANES + iota
            in_base = ibase + rows * NUM_FIELD
            cols = g * LANES + iota
            cols_c = [cols + c * CHUNK for c in range(EMB)]
            sbase = g * (NUM_CAT * LANES)

            colb = g * LANES
            for j in range(NUM_CONT):
                x = plsc.load_gather(in_v, [in_base + j])
                vss = [vs_v[pl.ds((j * EMB + c) * LANES, LANES)]
                       for c in range(EMB)]
                vals = [x * vs for vs in vss]
                for c in range(EMB):
                    out_v[j * EMB + c, pl.ds(colb, LANES)] = vals[c]

            c_true = jnp.zeros((LANES,), jnp.int32)
            c_false = jnp.zeros((LANES,), jnp.int32)
            for j in range(NUM_CONT, NUM_FIELD):
                x = plsc.load_gather(in_v, [in_base + j])
                mi = (x != 0.0).astype(jnp.int32)
                a = jnp.where(mi == 1, c_true + NUM_CONT, c_false)
                s = sbase + (j - NUM_CONT) * LANES
                as_v[pl.ds(s, LANES)] = a
                ms_v[pl.ds(s, LANES)] = mi
                c_true = c_true + mi
                c_false = c_false + (1 - mi)
            n_false = c_false

            for j in range(NUM_CONT, NUM_FIELD):
                s = sbase + (j - NUM_CONT) * LANES
                a = as_v[pl.ds(s, LANES)]
                mi = ms_v[pl.ds(s, LANES)]
                x = plsc.load_gather(in_v, [in_base + j])
                rank = a + (1 - mi) * (NUM_FIELD - n_false)
                scatter_16cols(x, rank * EMB, cols_c, j)
            return 0

        lax.fori_loop(0, GROUPS, group_body, 0)
        out_copy(ci).start()
        return 0

    lax.fori_loop(0, NCHUNKS, chunk_body, 0)
    out_copy(NCHUNKS - 1).wait()


def kernel(inputs, V):
    out_t = _emb_sc(inputs.reshape(-1), V[:NUM_FIELD].reshape(-1))
    return out_t.T.reshape(BATCH, NUM_FIELD, EMB)


# speedup vs baseline: 1.0166x; 1.0166x over previous
import functools

import jax
import jax.numpy as jnp
from jax import lax
from jax.experimental import pallas as pl
from jax.experimental.pallas import tpu as pltpu
from jax.experimental.pallas import tpu_sc as plsc

NUM_FIELD = 39
NUM_CONT = 13
NUM_CAT = NUM_FIELD - NUM_CONT
EMB = 16
BATCH = 16384
LANES = 16

NUM_CORES = 2
NUM_SUBCORES = 16
NW = NUM_CORES * NUM_SUBCORES
ROWS_PER_W = BATCH // NW
CHUNK = 128
NCHUNKS = ROWS_PER_W // CHUNK
GROUPS = CHUNK // LANES
OUT_ROWS = NUM_FIELD * EMB

_mesh = plsc.VectorSubcoreMesh(core_axis_name="c", subcore_axis_name="s")


@functools.partial(
    pl.kernel,
    mesh=_mesh,
    compiler_params=pltpu.CompilerParams(
        needs_layout_passes=False, use_tc_tiling_on_sc=True),
    out_type=jax.ShapeDtypeStruct((OUT_ROWS, BATCH), jnp.float32),
    scratch_types=[
        pltpu.VMEM((OUT_ROWS,), jnp.float32),
        pltpu.VMEM((OUT_ROWS * LANES,), jnp.float32),
        pltpu.VMEM((2 * CHUNK * NUM_FIELD,), jnp.float32),
        pltpu.VMEM((OUT_ROWS, CHUNK), jnp.float32),
        pltpu.VMEM((GROUPS * NUM_CAT * LANES,), jnp.int32),
        pltpu.VMEM((GROUPS * NUM_CAT * LANES,), jnp.int32),
        pltpu.SemaphoreType.DMA,
        pltpu.SemaphoreType.DMA,
    ],
)
def _emb_sc(in_hbm, v_hbm, out_hbm, v_v, vs_v, in_v, out_v, as_v, ms_v,
            in_sem, out_sem):
    wid = lax.axis_index("s") * NUM_CORES + lax.axis_index("c")
    iota = lax.iota(jnp.int32, LANES)
    pltpu.sync_copy(v_hbm, v_v)

    def splat_body(j, _):
        for c in range(EMB):
            sp = plsc.load_gather(
                v_v, [jnp.broadcast_to(j * EMB + c, (LANES,)).astype(jnp.int32)])
            plsc.store_scatter(vs_v, [(j * EMB + c) * LANES + iota], sp)
        return 0

    lax.fori_loop(0, NUM_FIELD, splat_body, 0)

    def in_copy(ci):
        base = wid * ROWS_PER_W + ci * CHUNK
        return pltpu.make_async_copy(
            in_hbm.at[pl.ds(base * NUM_FIELD, CHUNK * NUM_FIELD)],
            in_v.at[pl.ds((ci % 2) * (CHUNK * NUM_FIELD), CHUNK * NUM_FIELD)],
            in_sem)

    def out_copy(ci):
        base = wid * ROWS_PER_W + ci * CHUNK
        return pltpu.make_async_copy(
            out_v,
            out_hbm.at[:, pl.ds(base, CHUNK)],
            out_sem)

    def scatter_16cols(x, rowv, cols_c, j):
        vss = [vs_v[pl.ds((j * EMB + c) * LANES, LANES)] for c in range(EMB)]
        vals = [x * vs for vs in vss]
        for c in range(EMB):
            plsc.store_scatter(out_v, [rowv, cols_c[c]], vals[c])

    in_copy(0).start()

    def chunk_body(ci, _):
        @pl.when(ci + 1 < NCHUNKS)
        def _prefetch():
            in_copy(ci + 1).start()

        in_copy(ci).wait()

        @pl.when(ci >= 1)
        def _drain():
            out_copy(ci - 1).wait()

        ibase = (ci % 2) * (CHUNK * NUM_FIELD)

        def group_body(g, _):
            rows = g * LANES + iota
            in_base = ibase + rows * NUM_FIELD
            cols = g * LANES + iota
            cols_c = [cols + c * CHUNK for c in range(EMB)]
            sbase = g * (NUM_CAT * LANES)

            colb = g * LANES
            for j in range(NUM_CONT):
                x = plsc.load_gather(in_v, [in_base + j])
                vss = [vs_v[pl.ds((j * EMB + c) * LANES, LANES)]
                       for c in range(EMB)]
                vals = [x * vs for vs in vss]
                for c in range(EMB):
                    out_v[j * EMB + c, pl.ds(colb, LANES)] = vals[c]

            c_true = jnp.zeros((LANES,), jnp.int32)
            c_false = jnp.zeros((LANES,), jnp.int32)
            for j in range(NUM_CONT, NUM_FIELD):
                x = plsc.load_gather(in_v, [in_base + j])
                mi = (x != 0.0).astype(jnp.int32)
                a = jnp.where(mi == 1, c_true + NUM_CONT, c_false)
                s = sbase + (j - NUM_CONT) * LANES
                as_v[pl.ds(s, LANES)] = a
                ms_v[pl.ds(s, LANES)] = mi
                c_true = c_true + mi
                c_false = c_false + (1 - mi)
            n_false = c_false

            for j in range(NUM_CONT, NUM_FIELD):
                s = sbase + (j - NUM_CONT) * LANES
                a = as_v[pl.ds(s, LANES)]
                mi = ms_v[pl.ds(s, LANES)]
                x = plsc.load_gather(in_v, [in_base + j])
                rank = a + (1 - mi) * (NUM_FIELD - n_false)
                scatter_16cols(x, rank * EMB, cols_c, j)
            return 0

        lax.fori_loop(0, GROUPS, group_body, 0)
        out_copy(ci).start()
        return 0

    lax.fori_loop(0, NCHUNKS, chunk_body, 0)
    out_copy(NCHUNKS - 1).wait()


def kernel(inputs, V):
    out_t = _emb_sc(inputs.reshape(-1), V[:NUM_FIELD].reshape(-1))
    return out_t.T.reshape(BATCH, NUM_FIELD, EMB)


# speedup vs baseline: 1.1600x; 1.1411x over previous
import functools

import jax
import jax.numpy as jnp
from jax import lax
from jax.experimental import pallas as pl
from jax.experimental.pallas import tpu as pltpu
from jax.experimental.pallas import tpu_sc as plsc

NUM_FIELD = 39
NUM_CONT = 13
NUM_CAT = NUM_FIELD - NUM_CONT
EMB = 16
BATCH = 16384
LANES = 16

NUM_CORES = 2
NUM_SUBCORES = 16
NW = NUM_CORES * NUM_SUBCORES
ROWS_PER_W = BATCH // NW
CHUNK = 128
NCHUNKS = ROWS_PER_W // CHUNK
GROUPS = CHUNK // LANES
OUT_ROWS = NUM_FIELD * EMB

_mesh = plsc.VectorSubcoreMesh(core_axis_name="c", subcore_axis_name="s")


@functools.partial(
    pl.kernel,
    mesh=_mesh,
    compiler_params=pltpu.CompilerParams(
        needs_layout_passes=False, use_tc_tiling_on_sc=True),
    out_type=jax.ShapeDtypeStruct((OUT_ROWS, BATCH), jnp.float32),
    scratch_types=[
        pltpu.VMEM((OUT_ROWS,), jnp.float32),
        pltpu.VMEM((OUT_ROWS * LANES,), jnp.float32),
        pltpu.VMEM((NUM_FIELD, 2 * CHUNK), jnp.float32),
        pltpu.VMEM((OUT_ROWS, CHUNK), jnp.float32),
        pltpu.VMEM((GROUPS * NUM_CAT * LANES,), jnp.int32),
        pltpu.VMEM((GROUPS * NUM_CAT * LANES,), jnp.int32),
        pltpu.SemaphoreType.DMA,
        pltpu.SemaphoreType.DMA,
    ],
)
def _emb_sc(in_hbm, v_hbm, out_hbm, v_v, vs_v, in_v, out_v, as_v, ms_v,
            in_sem, out_sem):
    wid = lax.axis_index("s") * NUM_CORES + lax.axis_index("c")
    iota = lax.iota(jnp.int32, LANES)
    pltpu.sync_copy(v_hbm, v_v)

    def splat_body(j, _):
        for c in range(EMB):
            sp = plsc.load_gather(
                v_v, [jnp.broadcast_to(j * EMB + c, (LANES,)).astype(jnp.int32)])
            plsc.store_scatter(vs_v, [(j * EMB + c) * LANES + iota], sp)
        return 0

    lax.fori_loop(0, NUM_FIELD, splat_body, 0)

    def in_copy(ci):
        base = wid * ROWS_PER_W + ci * CHUNK
        return pltpu.make_async_copy(
            in_hbm.at[:, pl.ds(base, CHUNK)],
            in_v.at[:, pl.ds((ci % 2) * CHUNK, CHUNK)],
            in_sem)

    def out_copy(ci):
        base = wid * ROWS_PER_W + ci * CHUNK
        return pltpu.make_async_copy(
            out_v,
            out_hbm.at[:, pl.ds(base, CHUNK)],
            out_sem)

    def scatter_16cols(x, rowv, cols_c, j):
        vss = [vs_v[pl.ds((j * EMB + c) * LANES, LANES)] for c in range(EMB)]
        vals = [x * vs for vs in vss]
        for c in range(EMB):
            plsc.store_scatter(out_v, [rowv, cols_c[c]], vals[c])

    in_copy(0).start()

    def chunk_body(ci, _):
        @pl.when(ci + 1 < NCHUNKS)
        def _prefetch():
            in_copy(ci + 1).start()

        in_copy(ci).wait()

        @pl.when(ci >= 1)
        def _drain():
            out_copy(ci - 1).wait()

        ibase = (ci % 2) * CHUNK

        def group_body(g, _):
            icolb = ibase + g * LANES
            ocolb = g * LANES
            cols = g * LANES + iota
            cols_c = [cols + c * CHUNK for c in range(EMB)]
            sbase = g * (NUM_CAT * LANES)

            for j in range(NUM_CONT):
                x = in_v[j, pl.ds(icolb, LANES)]
                vss = [vs_v[pl.ds((j * EMB + c) * LANES, LANES)]
                       for c in range(EMB)]
                vals = [x * vs for vs in vss]
                for c in range(EMB):
                    out_v[j * EMB + c, pl.ds(ocolb, LANES)] = vals[c]

            c_true = jnp.zeros((LANES,), jnp.int32)
            c_false = jnp.zeros((LANES,), jnp.int32)
            for j in range(NUM_CONT, NUM_FIELD):
                x = in_v[j, pl.ds(icolb, LANES)]
                mi = (x != 0.0).astype(jnp.int32)
                a = jnp.where(mi == 1, c_true + NUM_CONT, c_false)
                s = sbase + (j - NUM_CONT) * LANES
                as_v[pl.ds(s, LANES)] = a
                ms_v[pl.ds(s, LANES)] = mi
                c_true = c_true + mi
                c_false = c_false + (1 - mi)
            n_false = c_false

            for j in range(NUM_CONT, NUM_FIELD):
                s = sbase + (j - NUM_CONT) * LANES
                a = as_v[pl.ds(s, LANES)]
                mi = ms_v[pl.ds(s, LANES)]
                x = in_v[j, pl.ds(icolb, LANES)]
                rank = a + (1 - mi) * (NUM_FIELD - n_false)
                scatter_16cols(x, rank * EMB, cols_c, j)
            return 0

        lax.fori_loop(0, GROUPS, group_body, 0)
        out_copy(ci).start()
        return 0

    lax.fori_loop(0, NCHUNKS, chunk_body, 0)
    out_copy(NCHUNKS - 1).wait()


def kernel(inputs, V):
    out_t = _emb_sc(inputs.T, V[:NUM_FIELD].reshape(-1))
    return out_t.T.reshape(BATCH, NUM_FIELD, EMB)
